# baseline (speedup 1.0000x reference)
import jax
import jax.numpy as jnp
from jax.experimental import pallas as pl
from jax.experimental.pallas import tpu as pltpu


def _celu(x, alpha=0.1):
    return jnp.where(x > 0, x, alpha * (jnp.exp(x / alpha) - 1.0))


def _make_body(BB):
    def _mlp_body(pss_ref, valid_ref, x_ref, w1_ref, b1_ref, w2_ref, b2_ref,
                  w3_ref, b3_ref, w4_ref, out_ref):
        j = pl.program_id(1)
        x = x_ref[...].reshape(2 * BB, x_ref.shape[2])
        h = _celu(jnp.dot(x, w1_ref[0], preferred_element_type=jnp.float32)
                  + b1_ref[0])
        h = _celu(jnp.dot(h.astype(jnp.bfloat16), w2_ref[0],
                          preferred_element_type=jnp.float32) + b2_ref[0])
        h = _celu(jnp.dot(h.astype(jnp.bfloat16), w3_ref[0],
                          preferred_element_type=jnp.float32) + b3_ref[0])
        y = jnp.dot(h.astype(jnp.bfloat16), w4_ref[0],
                    preferred_element_type=jnp.float32)
        y = y.reshape(2, BB, y.shape[1])
        f0 = valid_ref[2 * j].astype(jnp.float32)
        f1 = valid_ref[2 * j + 1].astype(jnp.float32)
        ys = y[0] * f0 + y[1] * f1

        @pl.when(j == 0)
        def _():
            out_ref[0] = ys

        @pl.when(j > 0)
        def _():
            out_ref[0] += ys

    return _mlp_body


def kernel(data, species, W1, b1, W2, b2, W3, b3, W4, b4):
    B, A, D = data.shape
    E, _, H1 = W1.shape
    H2 = W2.shape[2]
    H3 = W3.shape[2]
    LANE = 128

    species = species.astype(jnp.int32)
    order = jnp.argsort(species).astype(jnp.int32)

    A2 = A + E if (A + E) % 2 == 0 else A + E + 1
    c = jnp.sum(species[None, :] == jnp.arange(E)[:, None], axis=1)
    pe = c + (c % 2)
    poff = jnp.concatenate([jnp.zeros((1,), jnp.int32),
                            jnp.cumsum(pe).astype(jnp.int32)])
    off = jnp.concatenate([jnp.zeros((1,), jnp.int32),
                           jnp.cumsum(c).astype(jnp.int32)])
    p = jnp.arange(A2, dtype=jnp.int32)
    ep = jnp.clip(jnp.searchsorted(poff[1:], p, side="right"), 0, E - 1)
    r = p - poff[ep]
    valid = (r < c[ep]).astype(jnp.int32)
    src = jnp.clip(off[ep] + jnp.minimum(r, jnp.maximum(c[ep] - 1, 0)),
                   0, A - 1)
    pidx = order[src]
    pss = ep[::2].astype(jnp.int32)

    dataP = jnp.swapaxes(jnp.where(jnp.isnan(data), 0.0, data), 0, 1)
    dataP = dataP[pidx].astype(jnp.bfloat16)

    W1b = W1.astype(jnp.bfloat16)
    W2b = W2.astype(jnp.bfloat16)
    W3p = jnp.pad(W3, ((0, 0), (0, 0), (0, LANE - H3))).astype(jnp.bfloat16)
    w4p = jnp.pad(W4, ((0, 0), (0, LANE - H3),
                       (0, LANE - 1))).astype(jnp.bfloat16)
    b1r = b1.reshape(E, 1, H1)
    b2r = b2.reshape(E, 1, H2)
    b3r = jnp.pad(b3, ((0, 0), (0, LANE - H3))).reshape(E, 1, LANE)

    BB = 1024
    nb = B // BB
    grid = (nb, A2 // 2)

    def x_map(i, j, pss_ref, valid_ref):
        return (j, i, 0)

    def w_map(i, j, pss_ref, valid_ref):
        return (pss_ref[j], 0, 0)

    def out_map(i, j, pss_ref, valid_ref):
        return (i, 0, 0)

    out3 = pl.pallas_call(
        _make_body(BB),
        grid_spec=pltpu.PrefetchScalarGridSpec(
            num_scalar_prefetch=2,
            grid=grid,
            in_specs=[
                pl.BlockSpec((2, BB, D), x_map),
                pl.BlockSpec((1, D, H1), w_map),
                pl.BlockSpec((1, 1, H1), w_map),
                pl.BlockSpec((1, H1, H2), w_map),
                pl.BlockSpec((1, 1, H2), w_map),
                pl.BlockSpec((1, H2, LANE), w_map),
                pl.BlockSpec((1, 1, LANE), w_map),
                pl.BlockSpec((1, LANE, LANE), w_map),
            ],
            out_specs=pl.BlockSpec((1, BB, LANE), out_map),
        ),
        out_shape=jax.ShapeDtypeStruct((nb, BB, LANE), jnp.float32),
    )(pss, valid, dataP, W1b, b1r, W2b, b2r, W3p, b3r, w4p)

    out = out3[:, :, 0].reshape(B)
    out = out + jnp.sum(b4[species, 0])
    return out


# speedup vs baseline: 1.4484x; 1.4484x over previous
import jax
import jax.numpy as jnp
from jax.experimental import pallas as pl
from jax.experimental.pallas import tpu as pltpu


def _celu(x, alpha=0.1):
    return jnp.where(x > 0, x, alpha * (jnp.exp(x / alpha) - 1.0))


def _mlp_body(ss_ref, ord_ref, x_ref, w1_ref, b1_ref, w2_ref, b2_ref,
              w3_ref, b3_ref, w4_ref, out_ref):
    a = pl.program_id(1)
    x = x_ref[0]
    h = _celu(jnp.dot(x, w1_ref[0], preferred_element_type=jnp.float32)
              + b1_ref[0])
    h = _celu(jnp.dot(h.astype(jnp.bfloat16), w2_ref[0],
                      preferred_element_type=jnp.float32) + b2_ref[0])
    h = _celu(jnp.dot(h.astype(jnp.bfloat16), w3_ref[0],
                      preferred_element_type=jnp.float32) + b3_ref[0])
    y = jnp.dot(h.astype(jnp.bfloat16), w4_ref[0],
                preferred_element_type=jnp.float32)

    @pl.when(a == 0)
    def _():
        out_ref[0] = y

    @pl.when(a > 0)
    def _():
        out_ref[0] += y


def kernel(data, species, W1, b1, W2, b2, W3, b3, W4, b4):
    B, A, D = data.shape
    E, _, H1 = W1.shape
    H2 = W2.shape[2]
    H3 = W3.shape[2]
    LANE = 128

    species = species.astype(jnp.int32)
    order = jnp.argsort(species).astype(jnp.int32)
    ss = jnp.sort(species).astype(jnp.int32)

    dataT = jnp.swapaxes(jnp.where(jnp.isnan(data), 0.0, data), 0, 1)
    dataT = dataT.astype(jnp.bfloat16)

    W3p = jnp.pad(W3, ((0, 0), (0, 0), (0, LANE - H3)))
    b3p = jnp.pad(b3, ((0, 0), (0, LANE - H3))).reshape(E, 1, LANE)
    w4p = jnp.pad(W4, ((0, 0), (0, LANE - H3), (0, LANE - 1)))
    b1r = b1.reshape(E, 1, H1)
    b2r = b2.reshape(E, 1, H2)

    W1 = W1.astype(jnp.bfloat16)
    W2 = W2.astype(jnp.bfloat16)
    W3p = W3p.astype(jnp.bfloat16)
    w4p = w4p.astype(jnp.bfloat16)

    BB = 1024
    nb = B // BB
    grid = (nb, A)

    def x_map(i, a, ss_ref, ord_ref):
        return (ord_ref[a], i, 0)

    def w_map(i, a, ss_ref, ord_ref):
        return (ss_ref[a], 0, 0)

    def out_map(i, a, ss_ref, ord_ref):
        return (i, 0, 0)

    out3 = pl.pallas_call(
        _mlp_body,
        grid_spec=pltpu.PrefetchScalarGridSpec(
            num_scalar_prefetch=2,
            grid=grid,
            in_specs=[
                pl.BlockSpec((1, BB, D), x_map),
                pl.BlockSpec((1, D, H1), w_map),
                pl.BlockSpec((1, 1, H1), w_map),
                pl.BlockSpec((1, H1, H2), w_map),
                pl.BlockSpec((1, 1, H2), w_map),
                pl.BlockSpec((1, H2, LANE), w_map),
                pl.BlockSpec((1, 1, LANE), w_map),
                pl.BlockSpec((1, LANE, LANE), w_map),
            ],
            out_specs=pl.BlockSpec((1, BB, LANE), out_map),
        ),
        out_shape=jax.ShapeDtypeStruct((nb, BB, LANE), jnp.float32),
    )(ss, order, dataT, W1, b1r, W2, b2r, W3p, b3p, w4p)

    out = out3[:, :, 0].reshape(B)
    out = out + jnp.sum(b4[species, 0])
    return out
